# Initial kernel scaffold; baseline (speedup 1.0000x reference)
import jax
import jax.numpy as jnp
from jax.experimental import pallas as pl


def kernel(x, t1e, t1o, b1r, t2e, t2o, b2r, w1fc, b1fc, w2fc, b2fc):
    raise NotImplementedError("write your pallas kernel here")


# speedup vs baseline: 14.1266x; 14.1266x over previous
import jax
import jax.numpy as jnp
from jax.experimental import pallas as pl
from jax.experimental.pallas import tpu as pltpu


def _fused_kernel(x_ref, t1_ref, b1_ref, t2_ref, b2_ref,
                  w1_ref, bf1_ref, w2_ref, bf2_ref, o_ref):
    f32 = jnp.float32
    blk = o_ref.shape[0]
    m1 = 28 * blk

    xb = x_ref[...]
    zc = jnp.zeros((m1, 1), f32)
    xp = jnp.concatenate([zc, xb, zc], axis=1)
    zr = jnp.zeros((blk, 30), f32)
    up = jnp.concatenate([zr, xp[:-blk]], axis=0)
    dn = jnp.concatenate([xp[blk:], zr], axis=0)
    y = jnp.concatenate([up, xp, dn], axis=1)
    a = jnp.dot(y, t1_ref[...], preferred_element_type=f32)
    m = jnp.maximum(a[:, :256], a[:, 256:])
    m3 = m.reshape(14, 2 * blk, 256)
    mp = jnp.maximum(m3[:, :blk, :], m3[:, blk:, :])
    p1 = jnp.maximum(mp.reshape(14 * blk, 256) + b1_ref[...], 0.0)

    z2 = jnp.zeros((blk, 256), f32)
    up2 = jnp.concatenate([z2, p1[:-blk]], axis=0)
    dn2 = jnp.concatenate([p1[blk:], z2], axis=0)
    z = jnp.concatenate([up2, p1, dn2], axis=1)
    b = jnp.dot(z, t2_ref[...], preferred_element_type=f32)
    m2 = jnp.maximum(b[:, :256], b[:, 256:])
    m23 = m2.reshape(7, 2 * blk, 256)
    mp2 = jnp.maximum(m23[:, :blk, :], m23[:, blk:, :])
    p2 = jnp.maximum(mp2.reshape(7 * blk, 256) + b2_ref[...], 0.0)

    h = jnp.dot(p2[:blk], w1_ref[:256], preferred_element_type=f32)
    for hh in range(1, 7):
        h += jnp.dot(p2[hh * blk:(hh + 1) * blk],
                     w1_ref[hh * 256:(hh + 1) * 256],
                     preferred_element_type=f32)
    h = jnp.maximum(h + bf1_ref[...], 0.0)
    o_ref[...] = jnp.dot(h, w2_ref[...], preferred_element_type=f32) + bf2_ref[...]


def kernel(x, t1e, t1o, b1r, t2e, t2o, b2r, w1fc, b1fc, w2fc, b2fc):
    B = x.shape[0]
    blk = 128
    nb = -(-B // blk)
    Bp = nb * blk
    xs = x[:, 0]
    if Bp != B:
        xs = jnp.pad(xs, ((0, Bp - B), (0, 0), (0, 0)))
    xt = xs.reshape(nb, blk, 28, 28).transpose(0, 2, 1, 3).reshape(nb * 28 * blk, 28)
    t1c = jnp.concatenate([t1e.reshape(90, 256), t1o.reshape(90, 256)], axis=1)
    t2c = jnp.concatenate([t2e.reshape(768, 256), t2o.reshape(768, 256)], axis=1)
    out = pl.pallas_call(
        _fused_kernel,
        out_shape=jax.ShapeDtypeStruct((Bp, 128), jnp.float32),
        grid=(nb,),
        in_specs=[
            pl.BlockSpec((28 * blk, 28), lambda i: (i, 0)),
            pl.BlockSpec((90, 512), lambda i: (0, 0)),
            pl.BlockSpec((1, 256), lambda i: (0, 0)),
            pl.BlockSpec((768, 512), lambda i: (0, 0)),
            pl.BlockSpec((1, 256), lambda i: (0, 0)),
            pl.BlockSpec((1792, 128), lambda i: (0, 0)),
            pl.BlockSpec((1, 128), lambda i: (0, 0)),
            pl.BlockSpec((128, 128), lambda i: (0, 0)),
            pl.BlockSpec((1, 128), lambda i: (0, 0)),
        ],
        out_specs=pl.BlockSpec((blk, 128), lambda i: (i, 0)),
        compiler_params=pltpu.CompilerParams(dimension_semantics=("parallel",)),
    )(xt, t1c, b1r, t2c, b2r, w1fc, b1fc, w2fc, b2fc)
    return out[:B, :10]
